# baseline (speedup 1.0000x reference)
import functools
import math

import jax
import jax.numpy as jnp
from jax import lax
from jax.experimental import pallas as pl
from jax.experimental.pallas import tpu as pltpu
from jax.experimental.pallas import tpu_sc as plsc

_NBUF = 2
_LANES = 128


@functools.lru_cache(maxsize=None)
def _build(seq, bsz, embed, scale):
    info = plsc.get_sparse_core_info()
    nc, ns = info.num_cores, info.num_subcores
    nw = nc * ns
    nblk = bsz // _LANES
    assert nblk == nw and bsz == nblk * _LANES
    et = embed // 8
    ng = embed // 16
    assert embed % 16 == 0 and seq % _NBUF == 0 and seq >= 2 * _NBUF

    mesh = plsc.VectorSubcoreMesh(core_axis_name="c", subcore_axis_name="s")

    @functools.partial(
        pl.kernel,
        out_type=jax.ShapeDtypeStruct((seq, et, nblk, 8 * _LANES), jnp.float32),
        mesh=mesh,
        compiler_params=pltpu.CompilerParams(use_tc_tiling_on_sc=False,
                                             needs_layout_passes=False),
        scratch_types=[
            pltpu.VMEM((seq, _LANES), jnp.int32),
            pltpu.VMEM((seq, embed), jnp.float32),
            [pltpu.VMEM((_LANES, embed), jnp.float32) for _ in range(_NBUF)],
            [pltpu.VMEM((embed * _LANES,), jnp.float32) for _ in range(_NBUF)],
            [pltpu.SemaphoreType.DMA for _ in range(_NBUF)],
            [pltpu.SemaphoreType.DMA for _ in range(_NBUF)],
        ],
    )
    def emb_kernel(ids_hbm, tok_hbm, pos_hbm, out_hbm,
                   idx_v, pos_v, gbufs, obufs, gsems, ssems):
        w = lax.axis_index("s") * nc + lax.axis_index("c")

        pltpu.sync_copy(ids_hbm.at[:, pl.ds(w * _LANES, _LANES)], idx_v)
        pltpu.sync_copy(pos_hbm.at[pl.ds(0, seq)], pos_v)

        lane = lax.iota(jnp.int32, 16)

        def start_gather(b, s):
            pltpu.async_copy(tok_hbm.at[idx_v.at[s]], gbufs[b], gsems[b])

        def wait_gather(b, s):
            pltpu.make_async_copy(tok_hbm.at[idx_v.at[s]], gbufs[b],
                                  gsems[b]).wait()

        def start_store(b, s):
            for g8 in range(et):
                pltpu.async_copy(obufs[b].at[pl.ds(g8 * 8 * _LANES, 8 * _LANES)],
                                 out_hbm.at[s, g8, w], ssems[b])

        def wait_store(b, s):
            for g8 in range(et):
                pltpu.make_async_copy(obufs[b].at[pl.ds(g8 * 8 * _LANES, 8 * _LANES)],
                                      out_hbm.at[s, g8, w], ssems[b]).wait()

        def compute(b, s):
            for g in range(ng):
                pv = pos_v[s, pl.ds(g * 16, 16)]
                base = lane * _LANES + (g * 16 * _LANES)

                @pl.loop(0, _LANES)
                def _(j):
                    val = gbufs[b][j, pl.ds(g * 16, 16)] * scale + pv
                    plsc.store_scatter(obufs[b], [base + j], val)

        for b in range(_NBUF):
            start_gather(b, b)
        for b in range(_NBUF):
            wait_gather(b, b)
            compute(b, b)
            start_gather(b, b + _NBUF)
            start_store(b, b)

        @pl.loop(_NBUF, seq - _NBUF, step=_NBUF)
        def _(t):
            for b in range(_NBUF):
                s = t + b
                wait_gather(b, s)
                wait_store(b, s - _NBUF)
                compute(b, s)
                start_gather(b, s + _NBUF)
                start_store(b, s)

        for b in range(_NBUF):
            s = seq - _NBUF + b
            wait_gather(b, s)
            wait_store(b, s - _NBUF)
            compute(b, s)
            start_store(b, s)
        for b in range(_NBUF):
            wait_store(b, seq - _NBUF + b)

    return emb_kernel


def kernel(input_ids, key_padding_mask, token_table, pos_table):
    del key_padding_mask
    bsz, seq = input_ids.shape
    _, embed = token_table.shape
    ids_t = input_ids.astype(jnp.int32).T
    fn = _build(seq, bsz, embed, math.sqrt(embed))
    out4 = fn(ids_t, token_table, pos_table)
    out5 = out4.reshape(seq, embed // 8, bsz // 128, 8, 128)
    return jnp.transpose(out5, (2, 4, 0, 1, 3)).reshape(bsz, seq, embed)


# speedup vs baseline: 1.3361x; 1.3361x over previous
import functools
import math

import jax
import jax.numpy as jnp
from jax import lax
from jax.experimental import pallas as pl
from jax.experimental.pallas import tpu as pltpu
from jax.experimental.pallas import tpu_sc as plsc

_NBUF = 2
_LANES = 128


@functools.lru_cache(maxsize=None)
def _build(seq, bsz, embed, scale):
    info = plsc.get_sparse_core_info()
    nc, ns = info.num_cores, info.num_subcores
    nw = nc * ns
    nblk = bsz // _LANES
    assert nblk == nw and bsz == nblk * _LANES
    et = embed // 8
    ng = embed // 16
    assert embed % 16 == 0 and seq % _NBUF == 0 and seq >= 2 * _NBUF

    mesh = plsc.VectorSubcoreMesh(core_axis_name="c", subcore_axis_name="s")

    @functools.partial(
        pl.kernel,
        out_type=jax.ShapeDtypeStruct((seq, et, nblk, 8 * _LANES), jnp.float32),
        mesh=mesh,
        compiler_params=pltpu.CompilerParams(use_tc_tiling_on_sc=False,
                                             needs_layout_passes=False),
        scratch_types=[
            pltpu.VMEM((seq, _LANES), jnp.int32),
            pltpu.VMEM((seq, embed), jnp.float32),
            [pltpu.VMEM((_LANES, embed), jnp.float32) for _ in range(_NBUF)],
            [pltpu.VMEM((embed * _LANES,), jnp.float32) for _ in range(_NBUF)],
            [pltpu.SemaphoreType.DMA for _ in range(_NBUF)],
            [pltpu.SemaphoreType.DMA for _ in range(_NBUF)],
        ],
    )
    def emb_kernel(ids_hbm, tok_hbm, pos_hbm, out_hbm,
                   idx_v, pos_v, gbufs, obufs, gsems, ssems):
        w = lax.axis_index("s") * nc + lax.axis_index("c")

        pltpu.sync_copy(ids_hbm.at[:, pl.ds(w * _LANES, _LANES)], idx_v)
        pltpu.sync_copy(pos_hbm.at[pl.ds(0, seq)], pos_v)

        lane = lax.iota(jnp.int32, 16)

        def start_gather(b, s):
            pltpu.async_copy(tok_hbm.at[idx_v.at[s]], gbufs[b], gsems[b])

        def wait_gather(b, s):
            pltpu.make_async_copy(tok_hbm.at[idx_v.at[s]], gbufs[b],
                                  gsems[b]).wait()

        def start_store(b, s):
            for g8 in range(et):
                pltpu.async_copy(obufs[b].at[pl.ds(g8 * 8 * _LANES, 8 * _LANES)],
                                 out_hbm.at[s, g8, w], ssems[b])

        def wait_store(b, s):
            for g8 in range(et):
                pltpu.make_async_copy(obufs[b].at[pl.ds(g8 * 8 * _LANES, 8 * _LANES)],
                                      out_hbm.at[s, g8, w], ssems[b]).wait()

        def compute(b, s):
            for g in range(ng):
                pv = pos_v[s, pl.ds(g * 16, 16)]
                lane_e = lane + g * 16
                ebase = lane_e * _LANES

                @pl.loop(0, _LANES)
                def _(j0):
                    jm = lax.bitwise_and(j0 + lane, _LANES - 1)
                    val = plsc.load_gather(gbufs[b], [jm, lane_e])
                    plsc.store_scatter(obufs[b], [ebase + jm],
                                       val * scale + pv)

        for b in range(_NBUF):
            start_gather(b, b)
        for b in range(_NBUF):
            wait_gather(b, b)
            compute(b, b)
            start_gather(b, b + _NBUF)
            start_store(b, b)

        @pl.loop(_NBUF, seq - _NBUF, step=_NBUF)
        def _(t):
            for b in range(_NBUF):
                s = t + b
                wait_gather(b, s)
                wait_store(b, s - _NBUF)
                compute(b, s)
                start_gather(b, s + _NBUF)
                start_store(b, s)

        for b in range(_NBUF):
            s = seq - _NBUF + b
            wait_gather(b, s)
            wait_store(b, s - _NBUF)
            compute(b, s)
            start_store(b, s)
        for b in range(_NBUF):
            wait_store(b, seq - _NBUF + b)

    return emb_kernel


def kernel(input_ids, key_padding_mask, token_table, pos_table):
    del key_padding_mask
    bsz, seq = input_ids.shape
    _, embed = token_table.shape
    ids_t = input_ids.astype(jnp.int32).T
    fn = _build(seq, bsz, embed, math.sqrt(embed))
    out4 = fn(ids_t, token_table, pos_table)
    out5 = out4.reshape(seq, embed // 8, bsz // 128, 8, 128)
    return jnp.transpose(out5, (2, 4, 0, 1, 3)).reshape(bsz, seq, embed)


# speedup vs baseline: 1.4643x; 1.0959x over previous
import functools
import math

import jax
import jax.numpy as jnp
from jax import lax
from jax.experimental import pallas as pl
from jax.experimental.pallas import tpu as pltpu
from jax.experimental.pallas import tpu_sc as plsc

_NBUF = 2
_LANES = 128
_UNROLL = 4


@functools.lru_cache(maxsize=None)
def _build(seq, bsz, embed, scale):
    info = plsc.get_sparse_core_info()
    nc, ns = info.num_cores, info.num_subcores
    nw = nc * ns
    nblk = bsz // _LANES
    assert nblk == nw and bsz == nblk * _LANES
    et = embed // 8
    ng = embed // 16
    assert embed % 16 == 0 and seq % _NBUF == 0 and seq >= 2 * _NBUF

    mesh = plsc.VectorSubcoreMesh(core_axis_name="c", subcore_axis_name="s")

    @functools.partial(
        pl.kernel,
        out_type=jax.ShapeDtypeStruct((seq, et, nblk, 8 * _LANES), jnp.float32),
        mesh=mesh,
        compiler_params=pltpu.CompilerParams(use_tc_tiling_on_sc=False,
                                             needs_layout_passes=False),
        scratch_types=[
            pltpu.VMEM((seq, _LANES), jnp.int32),
            pltpu.VMEM((seq, embed), jnp.float32),
            [pltpu.VMEM((_LANES, embed), jnp.float32) for _ in range(_NBUF)],
            [pltpu.VMEM((embed * _LANES,), jnp.float32) for _ in range(_NBUF)],
            [pltpu.SemaphoreType.DMA for _ in range(_NBUF)],
            [pltpu.SemaphoreType.DMA for _ in range(_NBUF)],
        ],
    )
    def emb_kernel(ids_hbm, tok_hbm, pos_hbm, out_hbm,
                   idx_v, pos_v, gbufs, obufs, gsems, ssems):
        w = lax.axis_index("s") * nc + lax.axis_index("c")

        pltpu.sync_copy(ids_hbm.at[:, pl.ds(w * _LANES, _LANES)], idx_v)
        pltpu.sync_copy(pos_hbm.at[pl.ds(0, seq)], pos_v)

        lane = lax.iota(jnp.int32, 16)

        def start_gather(b, s):
            pltpu.async_copy(tok_hbm.at[idx_v.at[s]], gbufs[b], gsems[b])

        def wait_gather(b, s):
            pltpu.make_async_copy(tok_hbm.at[idx_v.at[s]], gbufs[b],
                                  gsems[b]).wait()

        def start_store(b, s):
            for g8 in range(et):
                pltpu.async_copy(obufs[b].at[pl.ds(g8 * 8 * _LANES, 8 * _LANES)],
                                 out_hbm.at[s, g8, w], ssems[b])

        def wait_store(b, s):
            for g8 in range(et):
                pltpu.make_async_copy(obufs[b].at[pl.ds(g8 * 8 * _LANES, 8 * _LANES)],
                                      out_hbm.at[s, g8, w], ssems[b]).wait()

        def compute(b, s):
            pvs = [pos_v[s, pl.ds(g * 16, 16)] for g in range(ng)]
            lane_es = [lane + g * 16 for g in range(ng)]
            ebases = [le * _LANES for le in lane_es]
            lane_us = [lane + u for u in range(_UNROLL)]

            @pl.loop(0, _LANES, step=_UNROLL)
            def _(j0):
                for u in range(_UNROLL):
                    jm = lax.bitwise_and(j0 + lane_us[u], _LANES - 1)
                    for g in range(ng):
                        val = plsc.load_gather(gbufs[b], [jm, lane_es[g]])
                        plsc.store_scatter(obufs[b], [ebases[g] + jm],
                                           val * scale + pvs[g])

        for b in range(_NBUF):
            start_gather(b, b)
        for b in range(_NBUF):
            wait_gather(b, b)
            compute(b, b)
            start_gather(b, b + _NBUF)
            start_store(b, b)

        @pl.loop(_NBUF, seq - _NBUF, step=_NBUF)
        def _(t):
            for b in range(_NBUF):
                s = t + b
                wait_gather(b, s)
                wait_store(b, s - _NBUF)
                compute(b, s)
                start_gather(b, s + _NBUF)
                start_store(b, s)

        for b in range(_NBUF):
            s = seq - _NBUF + b
            wait_gather(b, s)
            wait_store(b, s - _NBUF)
            compute(b, s)
            start_store(b, s)
        for b in range(_NBUF):
            wait_store(b, seq - _NBUF + b)

    return emb_kernel


def kernel(input_ids, key_padding_mask, token_table, pos_table):
    del key_padding_mask
    bsz, seq = input_ids.shape
    _, embed = token_table.shape
    ids_t = input_ids.astype(jnp.int32).T
    fn = _build(seq, bsz, embed, math.sqrt(embed))
    out4 = fn(ids_t, token_table, pos_table)
    out5 = out4.reshape(seq, embed // 8, bsz // 128, 8, 128)
    return jnp.transpose(out5, (2, 4, 0, 1, 3)).reshape(bsz, seq, embed)


# speedup vs baseline: 1.5027x; 1.0262x over previous
import functools
import math

import jax
import jax.numpy as jnp
from jax import lax
from jax.experimental import pallas as pl
from jax.experimental.pallas import tpu as pltpu
from jax.experimental.pallas import tpu_sc as plsc

_NBUF = 2
_LANES = 128
_UNROLL = 8


@functools.lru_cache(maxsize=None)
def _build(seq, bsz, embed, scale):
    info = plsc.get_sparse_core_info()
    nc, ns = info.num_cores, info.num_subcores
    nw = nc * ns
    nblk = bsz // _LANES
    assert nblk == nw and bsz == nblk * _LANES
    et = embed // 8
    ng = embed // 16
    assert embed % 16 == 0 and seq % _NBUF == 0 and seq >= 2 * _NBUF

    mesh = plsc.VectorSubcoreMesh(core_axis_name="c", subcore_axis_name="s")

    @functools.partial(
        pl.kernel,
        out_type=jax.ShapeDtypeStruct((seq, et, nblk, 8 * _LANES), jnp.float32),
        mesh=mesh,
        compiler_params=pltpu.CompilerParams(use_tc_tiling_on_sc=False,
                                             needs_layout_passes=False),
        scratch_types=[
            pltpu.VMEM((seq, _LANES), jnp.int32),
            pltpu.VMEM((seq, embed), jnp.float32),
            [pltpu.VMEM((_LANES, embed), jnp.float32) for _ in range(_NBUF)],
            [pltpu.VMEM((embed * _LANES,), jnp.float32) for _ in range(_NBUF)],
            [pltpu.SemaphoreType.DMA for _ in range(_NBUF)],
            [pltpu.SemaphoreType.DMA for _ in range(_NBUF)],
        ],
    )
    def emb_kernel(ids_hbm, tok_hbm, pos_hbm, out_hbm,
                   idx_v, pos_v, gbufs, obufs, gsems, ssems):
        w = lax.axis_index("s") * nc + lax.axis_index("c")

        pltpu.sync_copy(ids_hbm.at[:, pl.ds(w * _LANES, _LANES)], idx_v)
        pltpu.sync_copy(pos_hbm.at[pl.ds(0, seq)], pos_v)

        lane = lax.iota(jnp.int32, 16)

        def start_gather(b, s):
            pltpu.async_copy(tok_hbm.at[idx_v.at[s]], gbufs[b], gsems[b])

        def wait_gather(b, s):
            pltpu.make_async_copy(tok_hbm.at[idx_v.at[s]], gbufs[b],
                                  gsems[b]).wait()

        def start_store(b, s):
            for g8 in range(et):
                pltpu.async_copy(obufs[b].at[pl.ds(g8 * 8 * _LANES, 8 * _LANES)],
                                 out_hbm.at[s, g8, w], ssems[b])

        def wait_store(b, s):
            for g8 in range(et):
                pltpu.make_async_copy(obufs[b].at[pl.ds(g8 * 8 * _LANES, 8 * _LANES)],
                                      out_hbm.at[s, g8, w], ssems[b]).wait()

        def compute(b, s):
            pvs = [pos_v[s, pl.ds(g * 16, 16)] for g in range(ng)]
            lane_es = [lane + g * 16 for g in range(ng)]
            ebases = [le * _LANES for le in lane_es]
            lane_us = [lane + u for u in range(_UNROLL)]

            @pl.loop(0, _LANES, step=_UNROLL)
            def _(j0):
                for u in range(_UNROLL):
                    jm = lax.bitwise_and(j0 + lane_us[u], _LANES - 1)
                    for g in range(ng):
                        val = plsc.load_gather(gbufs[b], [jm, lane_es[g]])
                        plsc.store_scatter(obufs[b], [ebases[g] + jm],
                                           val * scale + pvs[g])

        for b in range(_NBUF):
            start_gather(b, b)
        for b in range(_NBUF):
            wait_gather(b, b)
            compute(b, b)
            start_gather(b, b + _NBUF)
            start_store(b, b)

        @pl.loop(_NBUF, seq - _NBUF, step=_NBUF)
        def _(t):
            for b in range(_NBUF):
                s = t + b
                wait_gather(b, s)
                wait_store(b, s - _NBUF)
                compute(b, s)
                start_gather(b, s + _NBUF)
                start_store(b, s)

        for b in range(_NBUF):
            s = seq - _NBUF + b
            wait_gather(b, s)
            wait_store(b, s - _NBUF)
            compute(b, s)
            start_store(b, s)
        for b in range(_NBUF):
            wait_store(b, seq - _NBUF + b)

    return emb_kernel


def kernel(input_ids, key_padding_mask, token_table, pos_table):
    del key_padding_mask
    bsz, seq = input_ids.shape
    _, embed = token_table.shape
    ids_t = input_ids.astype(jnp.int32).T
    fn = _build(seq, bsz, embed, math.sqrt(embed))
    out4 = fn(ids_t, token_table, pos_table)
    out5 = out4.reshape(seq, embed // 8, bsz // 128, 8, 128)
    return jnp.transpose(out5, (2, 4, 0, 1, 3)).reshape(bsz, seq, embed)
